# baseline (speedup 1.0000x reference)
import jax
import jax.numpy as jnp
from jax.experimental import pallas as pl
from jax.experimental.pallas import tpu as pltpu

_N = 8192
_D = 768
_E = 8
_H = 128
_GH = 64
_OUT = 768
_BAL = 0.01
_BLK = 1024
_BF = jnp.bfloat16
_F32 = jnp.float32


def _moe_body(x_ref, wg1_ref, bg1_ref, wg2_ref, bg2t_ref,
              w1_ref, b1_ref, w2_ref, b2_ref, w3_ref, b3_ref, pm_ref,
              out_ref, usage_ref, loss_ref,
              h2_ref, w1c_ref, w2b_ref, w3b_ref):
    i = pl.program_id(0)
    nblk = pl.num_programs(0)
    x = x_ref[...]

    @pl.when(i == 0)
    def _():
        for e in range(_E):
            w1c_ref[:, e * _H:(e + 1) * _H] = w1_ref[e].astype(_BF)
            w2b_ref[e] = w2_ref[e].astype(_BF)
        w3b_ref[...] = w3_ref[...].astype(_BF)
        usage_ref[...] = jnp.zeros_like(usage_ref)

    gh = jnp.maximum(jnp.dot(x, wg1_ref[...]) + bg1_ref[...], 0.0)
    logits = jnp.dot(gh, wg2_ref[...])

    xb = x.astype(_BF)
    h1 = jnp.maximum(
        jnp.dot(xb, w1c_ref[...], preferred_element_type=_F32)
        + b1_ref[...], 0.0).astype(_BF)
    for e in range(_E):
        h2_ref[:, e * _H:(e + 1) * _H] = jnp.maximum(
            jnp.dot(h1[:, e * _H:(e + 1) * _H], w2b_ref[e],
                    preferred_element_type=_F32) + b2_ref[e:e + 1],
            0.0).astype(_BF)

    lt = logits.T + bg2t_ref[...]

    eidx = jax.lax.broadcasted_iota(jnp.int32, lt.shape, 0)
    m1 = jnp.max(lt, axis=0, keepdims=True)
    i1 = jnp.min(jnp.where(lt == m1, eidx, _E), axis=0, keepdims=True)
    oh1 = eidx == i1
    rest = jnp.where(oh1, -jnp.inf, lt)
    m2 = jnp.max(rest, axis=0, keepdims=True)
    i2 = jnp.min(jnp.where(rest == m2, eidx, _E), axis=0, keepdims=True)
    oh2 = eidx == i2
    r = jnp.exp(m2 - m1)
    cwt = (jnp.where(oh1, 1.0, 0.0) + jnp.where(oh2, r, 0.0)) / (1.0 + r)

    cnt = jnp.sum((oh1 | oh2).astype(_F32), axis=1, keepdims=True)
    usage_ref[...] += cnt * (1.0 / _N)

    dn = (((0,), (0,)), ((), ()))
    cw_exp = jax.lax.dot_general(
        cwt.astype(_BF), pm_ref[...], dn, preferred_element_type=_F32)
    bias3 = jax.lax.dot_general(cwt, b3_ref[...], dn)

    out_ref[...] = jnp.dot(h2_ref[...] * cw_exp.astype(_BF), w3b_ref[...],
                           preferred_element_type=_F32) + bias3

    @pl.when(i == nblk - 1)
    def _():
        u = usage_ref[...]
        t = 1.0 / _E
        loss_ref[...] = jnp.mean((u - t) ** 2, keepdims=True) * _BAL


def kernel(x, Wg1, bg1, Wg2, bg2, W1, b1, W2, b2, W3, b3):
    grid = (_N // _BLK,)

    def fixed(shape):
        nd = len(shape)
        return pl.BlockSpec(shape, lambda i, _n=nd: (0,) * _n)

    b1c = b1.reshape(1, _E * _H)
    W3c = W3.reshape(_E * _H, _OUT)
    pmask = jnp.repeat(jnp.eye(_E, dtype=_BF), _H, axis=1)

    out, usage, loss = pl.pallas_call(
        _moe_body,
        grid=grid,
        in_specs=[
            pl.BlockSpec((_BLK, _D), lambda i: (i, 0)),
            fixed((_D, _GH)),
            fixed((1, _GH)),
            fixed((_GH, _E)),
            fixed((_E, 1)),
            fixed((_E, _D, _H)),
            fixed((1, _E * _H)),
            fixed((_E, _H, _H)),
            fixed((_E, _H)),
            fixed((_E * _H, _OUT)),
            fixed((_E, _OUT)),
            fixed((_E, _E * _H)),
        ],
        out_specs=[
            pl.BlockSpec((_BLK, _OUT), lambda i: (i, 0)),
            fixed((_E, 1)),
            fixed((1, 1)),
        ],
        out_shape=[
            jax.ShapeDtypeStruct((_N, _OUT), _F32),
            jax.ShapeDtypeStruct((_E, 1), _F32),
            jax.ShapeDtypeStruct((1, 1), _F32),
        ],
        scratch_shapes=[pltpu.VMEM((_BLK, _E * _H), _BF),
                        pltpu.VMEM((_D, _E * _H), _BF),
                        pltpu.VMEM((_E, _H, _H), _BF),
                        pltpu.VMEM((_E * _H, _OUT), _BF)],
        compiler_params=pltpu.CompilerParams(
            dimension_semantics=("arbitrary",),
        ),
    )(x, Wg1, bg1.reshape(1, _GH), Wg2, bg2.reshape(_E, 1),
      W1, b1c, W2, b2, W3c, b3, pmask)
    return out, loss[0, 0], usage[:, 0]


# speedup vs baseline: 1.0284x; 1.0284x over previous
import jax
import jax.numpy as jnp
from jax.experimental import pallas as pl
from jax.experimental.pallas import tpu as pltpu

_N = 8192
_D = 768
_E = 8
_H = 128
_GH = 64
_OUT = 768
_BAL = 0.01
_BLK = 1024


def _moe_body(x_ref, wg1_ref, bg1_ref, wg2_ref, bg2t_ref,
              w1_ref, b1_ref, w2_ref, b2_ref, w3_ref, b3_ref, pm_ref,
              out_ref, usage_ref, loss_ref, h2_ref, w1c_ref):
    i = pl.program_id(0)
    nblk = pl.num_programs(0)
    x = x_ref[...]

    @pl.when(i == 0)
    def _():
        for e in range(_E):
            w1c_ref[:, e * _H:(e + 1) * _H] = w1_ref[e]
        usage_ref[...] = jnp.zeros_like(usage_ref)

    gh = jnp.maximum(jnp.dot(x, wg1_ref[...]) + bg1_ref[...], 0.0)
    logits = jnp.dot(gh, wg2_ref[...])

    h1 = jnp.maximum(jnp.dot(x, w1c_ref[...]) + b1_ref[...], 0.0)
    for e in range(_E):
        h2_ref[:, e * _H:(e + 1) * _H] = jnp.maximum(
            jnp.dot(h1[:, e * _H:(e + 1) * _H], w2_ref[e]) + b2_ref[e:e + 1],
            0.0)

    lt = logits.T + bg2t_ref[...]

    eidx = jax.lax.broadcasted_iota(jnp.int32, lt.shape, 0)
    m1 = jnp.max(lt, axis=0, keepdims=True)
    i1 = jnp.min(jnp.where(lt == m1, eidx, _E), axis=0, keepdims=True)
    oh1 = eidx == i1
    rest = jnp.where(oh1, -jnp.inf, lt)
    m2 = jnp.max(rest, axis=0, keepdims=True)
    i2 = jnp.min(jnp.where(rest == m2, eidx, _E), axis=0, keepdims=True)
    oh2 = eidx == i2
    r = jnp.exp(m2 - m1)
    cwt = (jnp.where(oh1, 1.0, 0.0) + jnp.where(oh2, r, 0.0)) / (1.0 + r)

    cnt = jnp.sum((oh1 | oh2).astype(jnp.float32), axis=1, keepdims=True)
    usage_ref[...] += cnt * (1.0 / _N)

    dn = (((0,), (0,)), ((), ()))
    cw_exp = jax.lax.dot_general(
        cwt.astype(jnp.bfloat16), pm_ref[...], dn,
        preferred_element_type=jnp.float32)
    bias3 = jax.lax.dot_general(cwt, b3_ref[...], dn)

    out_ref[...] = jnp.dot(h2_ref[...] * cw_exp, w3_ref[...]) + bias3

    @pl.when(i == nblk - 1)
    def _():
        u = usage_ref[...]
        t = 1.0 / _E
        loss_ref[...] = jnp.mean((u - t) ** 2, keepdims=True) * _BAL


def kernel(x, Wg1, bg1, Wg2, bg2, W1, b1, W2, b2, W3, b3):
    grid = (_N // _BLK,)

    def fixed(shape):
        nd = len(shape)
        return pl.BlockSpec(shape, lambda i, _n=nd: (0,) * _n)

    b1c = b1.reshape(1, _E * _H)
    W3c = W3.reshape(_E * _H, _OUT)
    pmask = jnp.repeat(jnp.eye(_E, dtype=jnp.bfloat16), _H, axis=1)

    out, usage, loss = pl.pallas_call(
        _moe_body,
        grid=grid,
        in_specs=[
            pl.BlockSpec((_BLK, _D), lambda i: (i, 0)),
            fixed((_D, _GH)),
            fixed((1, _GH)),
            fixed((_GH, _E)),
            fixed((_E, 1)),
            fixed((_E, _D, _H)),
            fixed((1, _E * _H)),
            fixed((_E, _H, _H)),
            fixed((_E, _H)),
            fixed((_E * _H, _OUT)),
            fixed((_E, _OUT)),
            fixed((_E, _E * _H)),
        ],
        out_specs=[
            pl.BlockSpec((_BLK, _OUT), lambda i: (i, 0)),
            fixed((_E, 1)),
            fixed((1, 1)),
        ],
        out_shape=[
            jax.ShapeDtypeStruct((_N, _OUT), jnp.float32),
            jax.ShapeDtypeStruct((_E, 1), jnp.float32),
            jax.ShapeDtypeStruct((1, 1), jnp.float32),
        ],
        scratch_shapes=[pltpu.VMEM((_BLK, _E * _H), jnp.float32),
                        pltpu.VMEM((_D, _E * _H), jnp.float32)],
        compiler_params=pltpu.CompilerParams(
            dimension_semantics=("arbitrary",),
        ),
    )(x, Wg1, bg1.reshape(1, _GH), Wg2, bg2.reshape(_E, 1),
      W1, b1c, W2, b2, W3c, b3, pmask)
    return out, loss[0, 0], usage[:, 0]
